# baseline (speedup 1.0000x reference)
import jax
import jax.numpy as jnp
from jax.experimental import pallas as pl

_B = 4
_TS = 288
_DIM = 768
_K = int(_TS * _TS * 0.25)


def _gnn_body(at_ref, nodes_ref, w1_ref, b1_ref, w2_ref, b2_ref, wd_ref,
              bd_ref, out_ref):
    AT = at_ref[...]
    nodes = nodes_ref[...].reshape(_B * _TS, _DIM)

    bits = jax.lax.bitcast_convert_type(AT, jnp.int32)
    res = jnp.zeros((_B, 1, 1), jnp.int32)
    for bit in range(30, -1, -1):
        cand = res | jnp.int32(1 << bit)
        m = jnp.where(bits >= cand, jnp.int32(1), jnp.int32(0))
        cnt = jnp.sum(m, axis=(1, 2), keepdims=True)
        res = jnp.where(cnt >= _K, cand, res)

    S_T = jnp.where(bits >= res, AT, 0.0)
    deg = 1.0 + jnp.sum(S_T, axis=2, keepdims=True)
    dinv3 = 1.0 / jnp.sqrt(deg)
    dinv = dinv3.reshape(_B * _TS, 1)
    dinv2 = dinv * dinv
    S_Tb = S_T.astype(jnp.bfloat16)

    def gcn(hb, w_ref, b_ref):
        xw = jnp.dot(hb, w_ref[...], preferred_element_type=jnp.float32)
        y = (dinv * xw).reshape(_B, _TS, _DIM).astype(jnp.bfloat16)
        agg = jax.lax.dot_general(
            S_Tb, y, (((2,), (1,)), ((0,), (0,))),
            preferred_element_type=jnp.float32).reshape(_B * _TS, _DIM)
        return dinv * agg + dinv2 * xw + b_ref[...]

    h1 = jnp.maximum(gcn(nodes.astype(jnp.bfloat16), w1_ref, b1_ref), 0.0)
    h2 = jnp.maximum(gcn(h1.astype(jnp.bfloat16), w2_ref, b2_ref), 0.0)

    d = jnp.dot(h2, wd_ref[...], preferred_element_type=jnp.float32)
    p1 = 1.0 / (1.0 + jnp.exp(-(d + bd_ref[0, 0])))
    P = jnp.concatenate([1.0 - p1, p1], axis=1).reshape(_B, _TS, 2)
    out_ref[...] = jax.lax.dot_general(
        P, nodes.reshape(_B, _TS, _DIM), (((1,), (1,)), ((0,), (0,))),
        preferred_element_type=jnp.float32)


@jax.jit
def kernel(x, attn, W1, b1, W2, b2, Wc, bc):
    n = _TS
    non_skip_tk = x[:, 1:1 + n]
    skip_tk = x[:, 1 + n:]
    A_T = jnp.swapaxes(attn[:, 1 + n:, 1 + n:], 1, 2)

    wd = (Wc[:, 1] - Wc[:, 0]).reshape(_DIM, 1)
    bd = (bc[1] - bc[0]).reshape(1, 1)

    summaries = pl.pallas_call(
        _gnn_body,
        out_shape=jax.ShapeDtypeStruct((_B, 2, _DIM), jnp.float32),
    )(A_T, skip_tk, W1.astype(jnp.bfloat16), b1.reshape(1, _DIM),
      W2.astype(jnp.bfloat16), b2.reshape(1, _DIM), wd, bd)

    return jnp.concatenate([non_skip_tk, summaries], axis=1)


# speedup vs baseline: 1.0731x; 1.0731x over previous
import jax
import jax.numpy as jnp
from jax.experimental import pallas as pl
from jax.experimental.pallas import tpu as pltpu

_B = 4
_TS = 288
_DIM = 768
_K = int(_TS * _TS * 0.25)


def _gnn_body(at_ref, nodes_hbm, w1_hbm, w2_hbm, b1_ref, b2_ref, wd_ref,
              bd_ref, out_ref, nodes_v, w1_v, w2_v, sem_n, sem_w1, sem_w2):
    cp_n = pltpu.make_async_copy(nodes_hbm, nodes_v, sem_n)
    cp_w1 = pltpu.make_async_copy(w1_hbm, w1_v, sem_w1)
    cp_w2 = pltpu.make_async_copy(w2_hbm, w2_v, sem_w2)
    cp_n.start()
    cp_w1.start()
    cp_w2.start()

    AT = at_ref[...]

    bits = jax.lax.bitcast_convert_type(AT, jnp.int32)
    res = jnp.zeros((_B, 1, 1), jnp.int32)
    for bit in range(30, -1, -1):
        cand = res | jnp.int32(1 << bit)
        m = jnp.where(bits >= cand, jnp.int32(1), jnp.int32(0))
        cnt = jnp.sum(m, axis=(1, 2), keepdims=True)
        res = jnp.where(cnt >= _K, cand, res)

    S_T = jnp.where(bits >= res, AT, 0.0)
    deg = 1.0 + jnp.sum(S_T, axis=2, keepdims=True)
    dinv3 = 1.0 / jnp.sqrt(deg)
    dinv = dinv3.reshape(_B * _TS, 1)
    dinv2 = dinv * dinv

    cp_n.wait()
    cp_w1.wait()
    nodes = nodes_v[...].reshape(_B * _TS, _DIM)

    def gcn(h, w_v, b_ref):
        xw = jnp.dot(h, w_v[...], preferred_element_type=jnp.float32)
        y = (dinv * xw).reshape(_B, _TS, _DIM)
        agg = jax.lax.dot_general(
            S_T, y, (((2,), (1,)), ((0,), (0,))),
            preferred_element_type=jnp.float32).reshape(_B * _TS, _DIM)
        return dinv * agg + dinv2 * xw + b_ref[...]

    h1 = jnp.maximum(gcn(nodes, w1_v, b1_ref), 0.0)
    cp_w2.wait()
    h2 = jnp.maximum(gcn(h1, w2_v, b2_ref), 0.0)

    d = jnp.sum(h2 * wd_ref[...], axis=1, keepdims=True) + bd_ref[0, 0]
    p1 = 1.0 / (1.0 + jnp.exp(-d))
    p0 = 1.0 - p1
    nodes3 = nodes.reshape(_B, _TS, _DIM)
    r0 = jnp.sum(p0.reshape(_B, _TS, 1) * nodes3, axis=1, keepdims=True)
    r1 = jnp.sum(p1.reshape(_B, _TS, 1) * nodes3, axis=1, keepdims=True)
    out_ref[...] = jnp.concatenate([r0, r1], axis=1)


@jax.jit
def kernel(x, attn, W1, b1, W2, b2, Wc, bc):
    n = _TS
    non_skip_tk = x[:, 1:1 + n]
    skip_tk = x[:, 1 + n:]
    A_T = jnp.swapaxes(attn[:, 1 + n:, 1 + n:], 1, 2)

    wd = (Wc[:, 1] - Wc[:, 0]).reshape(1, _DIM)
    bd = (bc[1] - bc[0]).reshape(1, 1)

    summaries = pl.pallas_call(
        _gnn_body,
        in_specs=[
            pl.BlockSpec(memory_space=pltpu.MemorySpace.VMEM),
            pl.BlockSpec(memory_space=pltpu.MemorySpace.HBM),
            pl.BlockSpec(memory_space=pltpu.MemorySpace.HBM),
            pl.BlockSpec(memory_space=pltpu.MemorySpace.HBM),
            pl.BlockSpec(memory_space=pltpu.MemorySpace.VMEM),
            pl.BlockSpec(memory_space=pltpu.MemorySpace.VMEM),
            pl.BlockSpec(memory_space=pltpu.MemorySpace.VMEM),
            pl.BlockSpec(memory_space=pltpu.MemorySpace.VMEM),
        ],
        out_specs=pl.BlockSpec(memory_space=pltpu.MemorySpace.VMEM),
        out_shape=jax.ShapeDtypeStruct((_B, 2, _DIM), jnp.float32),
        scratch_shapes=[
            pltpu.VMEM((_B, _TS, _DIM), jnp.float32),
            pltpu.VMEM((_DIM, _DIM), jnp.float32),
            pltpu.VMEM((_DIM, _DIM), jnp.float32),
            pltpu.SemaphoreType.DMA,
            pltpu.SemaphoreType.DMA,
            pltpu.SemaphoreType.DMA,
        ],
    )(A_T, skip_tk, W1, W2, b1.reshape(1, _DIM), b2.reshape(1, _DIM), wd, bd)

    return jnp.concatenate([non_skip_tk, summaries], axis=1)
